# Initial kernel scaffold; baseline (speedup 1.0000x reference)
import jax
import jax.numpy as jnp
from jax.experimental import pallas as pl


def kernel(time_idx, tp_idx, feats, dpcs_mark, edge_index_comp, edge_feat_comp, edge_index_coop, edge_feat_coop, time_table, tp_table, cs_table, Wg1, a_src1, a_dst1, a_e1, Wg2, a_src2, a_dst2, a_e2, W_p, b_p, v_f, v_d):
    raise NotImplementedError("write your pallas kernel here")


# speedup vs baseline: 4.9366x; 4.9366x over previous
import functools

import jax
import jax.numpy as jnp
from jax import lax
from jax.experimental import pallas as pl
from jax.experimental.pallas import tpu as pltpu
from jax.experimental.pallas import tpu_sc as plsc

B = 16
N = 3125
NUM_NODES = B * N
E = 1600000
COM = 16
HID = 64
NEG = -1e8
TOPK = int(N * 0.1 + 0.5)

RB = 1000
NRB = NUM_NODES // RB
EROW = 128
NROWS = E // EROW
NWORK = 32
TPS = 3128
NUM_PAD = 16 * TPS
DTS = 3200
DEN_PAD = 16 * DTS

_f32 = jnp.float32
_i32 = jnp.int32


def _embed_body(tid_ref, pid_ref, cs_ref, ft_ref, tt_ref, pt_ref, wg_ref,
                as_ref, ad_ref, x_ref, h_ref, hs_ref, hd_ref):
    tid = tid_ref[0, 0, :]
    pid = pid_ref[0, 0, :]
    t_oh = (tid[:, None] == lax.broadcasted_iota(_i32, (RB, 48), 1)).astype(_f32)
    p_oh = (pid[:, None] == lax.broadcasted_iota(_i32, (RB, 4), 1)).astype(_f32)
    t_emb = jnp.dot(t_oh, tt_ref[...], preferred_element_type=_f32)
    tp_e = jnp.dot(p_oh, pt_ref[...], preferred_element_type=_f32)
    x = jnp.concatenate([cs_ref[0], tp_e, ft_ref[0]], axis=1)
    h = jnp.dot(x, wg_ref[...], preferred_element_type=_f32)
    x_ref[0] = x
    h_ref[0] = h
    hs_ref[0] = jnp.dot(h, as_ref[...], preferred_element_type=_f32)
    hd_ref[0] = jnp.dot(h, ad_ref[...], preferred_element_type=_f32)


def _embed_call(tid, pid, cs_b, ft, time_table, tp_table, Wg1, a_s, a_d):
    blk = lambda *shape: pl.BlockSpec(shape, lambda i: (i,) + (0,) * (len(shape) - 1))
    full = lambda arr: pl.BlockSpec(arr.shape, lambda i: (0,) * arr.ndim)
    return pl.pallas_call(
        _embed_body,
        grid=(NRB,),
        in_specs=[blk(1, 1, RB), blk(1, 1, RB), blk(1, RB, 4), blk(1, RB, 9),
                  full(time_table), full(tp_table), full(Wg1), full(a_s),
                  full(a_d)],
        out_specs=[blk(1, RB, 15), blk(1, RB, COM), blk(1, RB, 1), blk(1, RB, 1)],
        out_shape=[jax.ShapeDtypeStruct((NRB, RB, 15), _f32),
                   jax.ShapeDtypeStruct((NRB, RB, COM), _f32),
                   jax.ShapeDtypeStruct((NRB, RB, 1), _f32),
                   jax.ShapeDtypeStruct((NRB, RB, 1), _f32)],
    )(tid, pid, cs_b, ft, time_table, tp_table, Wg1, a_s, a_d)


_EB = 2000
_NEB = E // _EB


def _ee_body(ef1_ref, ef2_ref, ae1_ref, ae2_ref, o1_ref, o2_ref):
    o1_ref[0] = jnp.dot(ef1_ref[0], ae1_ref[...], preferred_element_type=_f32)
    o2_ref[0] = jnp.dot(ef2_ref[0], ae2_ref[...], preferred_element_type=_f32)


def _ee_call(ef1, ef2, ae1, ae2):
    blk = pl.BlockSpec((1, _EB, 3), lambda i: (i, 0, 0))
    fullv = pl.BlockSpec((3, 1), lambda i: (0, 0))
    oblk = pl.BlockSpec((1, _EB, 1), lambda i: (i, 0, 0))
    return pl.pallas_call(
        _ee_body,
        grid=(_NEB,),
        in_specs=[blk, blk, fullv, fullv],
        out_specs=[oblk, oblk],
        out_shape=[jax.ShapeDtypeStruct((_NEB, _EB, 1), _f32),
                   jax.ShapeDtypeStruct((_NEB, _EB, 1), _f32)],
    )(ef1, ef2, ae1, ae2)


def _edge_body(src_hbm, dst_hbm, ee_hbm, h_hbm, hs_hbm, hd_hbm,
               num_out, den_out, num_sh, den_sh,
               src_v, dst_v, ee_v, hs_v, hd_v, hrows, w_v, out_v,
               zbuf, zbufd):
    c = lax.axis_index("c")
    s = lax.axis_index("s")
    wid = c * 16 + s

    def _zrow(i, _):
        zbuf[i, :] = jnp.zeros((16,), _f32)
        return 0
    lax.fori_loop(0, TPS, _zrow, 0)

    def _zd(i, _):
        zbufd[pl.ds(i * 16, 16)] = jnp.zeros((16,), _f32)
        return 0
    lax.fori_loop(0, DTS // 16, _zd, 0)

    nbase = s * TPS
    dbase = s * DTS
    pltpu.sync_copy(zbuf, num_sh.at[pl.ds(nbase, TPS)])
    pltpu.sync_copy(zbufd, den_sh.at[pl.ds(dbase, DTS)])
    plsc.subcore_barrier()

    r0 = wid * NROWS // NWORK
    r1 = (wid + 1) * NROWS // NWORK

    def _row(r, _):
        pltpu.sync_copy(src_hbm.at[r], src_v)
        pltpu.sync_copy(dst_hbm.at[r], dst_v)
        pltpu.sync_copy(ee_hbm.at[r], ee_v)
        pltpu.sync_copy(hs_hbm.at[src_v], hs_v)
        pltpu.sync_copy(hd_hbm.at[dst_v], hd_v)
        pltpu.sync_copy(h_hbm.at[src_v], hrows)
        for g in range(EROW // 16):
            off = g * 16
            t = hs_v[pl.ds(off, 16)] + hd_v[pl.ds(off, 16)] + ee_v[pl.ds(off, 16)]
            t = jnp.where(t >= 0.0, t, 0.2 * t)
            w = jnp.exp(t)
            w_v[pl.ds(off, 16)] = w
            for j in range(16):
                e = off + j
                out_v[e, :] = hrows[e, :] * w[j]
        pltpu.sync_copy(out_v, num_sh.at[dst_v], add=True)
        pltpu.sync_copy(w_v, den_sh.at[dst_v], add=True)
        return 0

    lax.fori_loop(r0, r1, _row, 0)
    plsc.subcore_barrier()

    pltpu.sync_copy(num_sh.at[pl.ds(nbase, TPS)],
                    num_out.at[c, pl.ds(nbase, TPS)])
    pltpu.sync_copy(den_sh.at[pl.ds(dbase, DTS)],
                    den_out.at[c, pl.ds(dbase, DTS)])


def _edge_pass(src_r, dst_r, ee_r, h, hs, hd):
    h = jnp.concatenate([h, jnp.zeros((NUM_PAD - NUM_NODES, COM), _f32)])
    mesh = plsc.VectorSubcoreMesh(core_axis_name="c", subcore_axis_name="s",
                                  num_cores=2, num_subcores=16)
    return pl.kernel(
        _edge_body,
        out_type=[jax.ShapeDtypeStruct((2, NUM_PAD, COM), _f32),
                  jax.ShapeDtypeStruct((2, DEN_PAD), _f32)],
        mesh=mesh,
        scratch_types=[
            pltpu.VMEM_SHARED((NUM_PAD, COM), _f32),
            pltpu.VMEM_SHARED((DEN_PAD,), _f32),
            pltpu.VMEM((EROW,), _i32),
            pltpu.VMEM((EROW,), _i32),
            pltpu.VMEM((EROW,), _f32),
            pltpu.VMEM((EROW,), _f32),
            pltpu.VMEM((EROW,), _f32),
            pltpu.VMEM((EROW, COM), _f32),
            pltpu.VMEM((EROW,), _f32),
            pltpu.VMEM((EROW, COM), _f32),
            pltpu.VMEM((TPS, 16), _f32),
            pltpu.VMEM((DTS,), _f32),
        ],
        compiler_params=pltpu.CompilerParams(use_tc_tiling_on_sc=False),
    )(src_r, dst_r, ee_r, h, hs, hd)


def _mid_body(n0_ref, n1_ref, d0_ref, d1_ref, x_ref, wg_ref, as_ref, ad_ref,
              comp_ref, h_ref, hs_ref, hd_ref):
    num = n0_ref[0] + n1_ref[0]
    den = d0_ref[0] + d1_ref[0] + 1e-9
    agg = num / den
    comp = jnp.where(agg > 0.0, agg, jnp.exp(jnp.minimum(agg, 0.0)) - 1.0)
    xm = jnp.concatenate([x_ref[0], comp], axis=1)
    h = jnp.dot(xm, wg_ref[...], preferred_element_type=_f32)
    comp_ref[0] = comp
    h_ref[0] = h
    hs_ref[0] = jnp.dot(h, as_ref[...], preferred_element_type=_f32)
    hd_ref[0] = jnp.dot(h, ad_ref[...], preferred_element_type=_f32)


def _mid_call(n0, n1, d0, d1, x, Wg2, a_s, a_d):
    blk = lambda *shape: pl.BlockSpec(shape, lambda i: (i,) + (0,) * (len(shape) - 1))
    full = lambda arr: pl.BlockSpec(arr.shape, lambda i: (0,) * arr.ndim)
    return pl.pallas_call(
        _mid_body,
        grid=(NRB,),
        in_specs=[blk(1, RB, COM), blk(1, RB, COM), blk(1, RB, 1), blk(1, RB, 1),
                  blk(1, RB, 15), full(Wg2), full(a_s), full(a_d)],
        out_specs=[blk(1, RB, COM), blk(1, RB, COM), blk(1, RB, 1), blk(1, RB, 1)],
        out_shape=[jax.ShapeDtypeStruct((NRB, RB, COM), _f32),
                   jax.ShapeDtypeStruct((NRB, RB, COM), _f32),
                   jax.ShapeDtypeStruct((NRB, RB, 1), _f32),
                   jax.ShapeDtypeStruct((NRB, RB, 1), _f32)],
    )(n0, n1, d0, d1, x, Wg2, a_s, a_d)


def _xp_body(n0_ref, n1_ref, d0_ref, d1_ref, x_ref, comp_ref, tid_ref, tt_ref,
             wp_ref, bp_ref, xp_ref):
    num = n0_ref[0] + n1_ref[0]
    den = d0_ref[0] + d1_ref[0] + 1e-9
    agg = num / den
    coop = jnp.where(agg > 0.0, agg, jnp.exp(jnp.minimum(agg, 0.0)) - 1.0)
    tid = tid_ref[0, 0, :]
    t_oh = (tid[:, None] == lax.broadcasted_iota(_i32, (RB, 48), 1)).astype(_f32)
    t_emb = jnp.dot(t_oh, tt_ref[...], preferred_element_type=_f32)
    sagnn = jnp.concatenate([t_emb, x_ref[0], comp_ref[0], coop], axis=1)
    xp_ref[0] = jnp.dot(sagnn, wp_ref[...],
                        preferred_element_type=_f32) + bp_ref[...]


def _xp_call(n0, n1, d0, d1, x, comp, tid, time_table, W_p, b_p):
    blk = lambda *shape: pl.BlockSpec(shape, lambda i: (i,) + (0,) * (len(shape) - 1))
    full = lambda arr: pl.BlockSpec(arr.shape, lambda i: (0,) * arr.ndim)
    return pl.pallas_call(
        _xp_body,
        grid=(NRB,),
        in_specs=[blk(1, RB, COM), blk(1, RB, COM), blk(1, RB, 1), blk(1, RB, 1),
                  blk(1, RB, 15), blk(1, RB, COM), blk(1, 1, RB),
                  full(time_table), full(W_p), full(b_p)],
        out_specs=[blk(1, RB, HID)],
        out_shape=[jax.ShapeDtypeStruct((NRB, RB, HID), _f32)],
    )(n0, n1, d0, d1, x, comp, tid, time_table, W_p, b_p)[0]


def _gate_one(xp, s, keep):
    s = s - jnp.max(s)
    s = jnp.where(keep, s, NEG)
    bits = lax.bitcast_convert_type(s, _i32)
    u = jnp.where(bits < 0, ~bits,
                  bits ^ jnp.int32(-2147483648)).astype(jnp.uint32)

    def _bit(i, ans):
        cand = ans | (jnp.uint32(1) << (jnp.uint32(31) - i.astype(jnp.uint32)))
        cnt = jnp.sum((u >= cand).astype(_i32))
        return jnp.where(cnt >= TOPK, cand, ans)

    thr = lax.fori_loop(0, 32, _bit, jnp.uint32(0))
    s = jnp.where(u < thr, NEG, s)
    m = jnp.max(s)
    ex = jnp.exp(s - m)
    w = ex / jnp.sum(ex)
    xg = xp * w
    return (jnp.sum(xg, axis=0, keepdims=True),
            jnp.max(xg, axis=0, keepdims=True))


def _gate_body(xp_ref, mark_ref, vf_ref, vd_ref, out_ref):
    xp = xp_ref[0]
    mark = mark_ref[0]
    sd = jnp.dot(xp, vd_ref[...], preferred_element_type=_f32)
    sf = jnp.dot(xp, vf_ref[...], preferred_element_type=_f32)
    fsum, fmax = _gate_one(xp, sf, mark < 1e-8)
    dsum, dmax = _gate_one(xp, sd, mark > 1e-8)
    out_ref[0, :, pl.ds(0, HID)] = fsum
    out_ref[0, :, pl.ds(HID, HID)] = fmax
    out_ref[0, :, pl.ds(2 * HID, HID)] = dsum
    out_ref[0, :, pl.ds(3 * HID, HID)] = dmax


def _gate_call(xp, mark, v_f, v_d):
    full = lambda arr: pl.BlockSpec(arr.shape, lambda i: (0,) * arr.ndim)
    return pl.pallas_call(
        _gate_body,
        grid=(B,),
        in_specs=[pl.BlockSpec((1, N, HID), lambda i: (i, 0, 0)),
                  pl.BlockSpec((1, N, 1), lambda i: (i, 0, 0)),
                  full(v_f), full(v_d)],
        out_specs=pl.BlockSpec((1, 1, 4 * HID), lambda i: (i, 0, 0)),
        out_shape=jax.ShapeDtypeStruct((B, 1, 4 * HID), _f32),
    )(xp, mark, v_f, v_d)


def kernel(time_idx, tp_idx, feats, dpcs_mark, edge_index_comp, edge_feat_comp,
           edge_index_coop, edge_feat_coop, time_table, tp_table, cs_table,
           Wg1, a_src1, a_dst1, a_e1, Wg2, a_src2, a_dst2, a_e2,
           W_p, b_p, v_f, v_d):
    tid3 = time_idx.reshape(NRB, 1, RB)
    pid3 = tp_idx.reshape(NRB, 1, RB)
    ft = feats.reshape(NRB, RB, 9)
    cs_b = jnp.broadcast_to(cs_table[None], (B, N, 4)).reshape(NRB, RB, 4)

    x, h1, hs1, hd1 = _embed_call(
        tid3, pid3, cs_b, ft, time_table, tp_table, Wg1,
        a_src1.reshape(COM, 1), a_dst1.reshape(COM, 1))

    ee1, ee2 = _ee_call(edge_feat_comp.reshape(_NEB, _EB, 3),
                        edge_feat_coop.reshape(_NEB, _EB, 3),
                        a_e1.reshape(3, 1), a_e2.reshape(3, 1))

    src1 = edge_index_comp[0].reshape(NROWS, EROW)
    dst1 = edge_index_comp[1].reshape(NROWS, EROW)
    num1, den1 = _edge_pass(src1, dst1, ee1.reshape(NROWS, EROW),
                            h1.reshape(NUM_NODES, COM),
                            hs1.reshape(NUM_NODES), hd1.reshape(NUM_NODES))

    comp, h2, hs2, hd2 = _mid_call(
        num1[0, :NUM_NODES].reshape(NRB, RB, COM),
        num1[1, :NUM_NODES].reshape(NRB, RB, COM),
        den1[0, :NUM_NODES].reshape(NRB, RB, 1),
        den1[1, :NUM_NODES].reshape(NRB, RB, 1),
        x, Wg2, a_src2.reshape(COM, 1), a_dst2.reshape(COM, 1))

    src2 = edge_index_coop[0].reshape(NROWS, EROW)
    dst2 = edge_index_coop[1].reshape(NROWS, EROW)
    num2, den2 = _edge_pass(src2, dst2, ee2.reshape(NROWS, EROW),
                            h2.reshape(NUM_NODES, COM),
                            hs2.reshape(NUM_NODES), hd2.reshape(NUM_NODES))

    xp = _xp_call(
        num2[0, :NUM_NODES].reshape(NRB, RB, COM),
        num2[1, :NUM_NODES].reshape(NRB, RB, COM),
        den2[0, :NUM_NODES].reshape(NRB, RB, 1),
        den2[1, :NUM_NODES].reshape(NRB, RB, 1),
        x, comp, tid3, time_table, W_p, b_p.reshape(1, HID))

    out = _gate_call(xp.reshape(B, N, HID), dpcs_mark,
                     v_f.reshape(HID, 1), v_d.reshape(HID, 1))
    return out.reshape(B, 4 * HID)
